# baseline (speedup 1.0000x reference)
import functools

import jax
import jax.numpy as jnp
from jax import lax
from jax.experimental import pallas as pl
from jax.experimental.pallas import tpu as pltpu
from jax.experimental.pallas import tpu_sc as plsc

CHUNK = 400
NBUF = 8


def _build(N, V, D, n_per_w, num_cores):
    mesh = plsc.VectorSubcoreMesh(core_axis_name="c", subcore_axis_name="s")
    n_chunks = n_per_w // CHUNK
    assert n_chunks % NBUF == 0 and n_chunks >= 2 * NBUF

    @functools.partial(
        pl.kernel,
        mesh=mesh,
        out_type=jax.ShapeDtypeStruct((N, D), jnp.float32),
        scratch_types=[
            pltpu.VMEM((n_per_w,), jnp.int32),
            pltpu.VMEM((NBUF, CHUNK, D), jnp.float32),
            pltpu.SemaphoreType.DMA,
            pltpu.SemaphoreType.DMA,
        ],
        compiler_params=pltpu.CompilerParams(use_tc_tiling_on_sc=False),
    )
    def k(x_hbm, table_hbm, out_hbm, idx_v, rows_v, gsem, osem):
        wid = lax.axis_index("s") * num_cores + lax.axis_index("c")
        base = wid * n_per_w

        pltpu.sync_copy(x_hbm.at[pl.ds(base, n_per_w)], idx_v)

        def gather(i, b):
            return pltpu.async_copy(
                table_hbm.at[idx_v.at[pl.ds(i * CHUNK, CHUNK)]],
                rows_v.at[b],
                gsem,
            )

        def writeback(i, b):
            return pltpu.async_copy(
                rows_v.at[b],
                out_hbm.at[pl.ds(base + i * CHUNK, CHUNK), :],
                osem,
            )

        def drain_gather(b):
            pltpu.make_async_copy(
                table_hbm.at[idx_v.at[pl.ds(0, CHUNK)]], rows_v.at[b], gsem
            ).wait()

        def drain_write(b):
            pltpu.make_async_copy(
                rows_v.at[b], out_hbm.at[pl.ds(base, CHUNK), :], osem
            ).wait()

        for b in range(NBUF):
            gather(b, b)

        def body(g, carry):
            for b in range(NBUF):
                i = g + b
                drain_gather(b)
                writeback(i, b)
                drain_write(b)
                gather(i + NBUF, b)
            return carry

        lax.fori_loop(0, (n_chunks - NBUF) // NBUF, lambda t, c: body(t * NBUF, c), 0)

        for b in range(NBUF):
            i = n_chunks - NBUF + b
            drain_gather(b)
            writeback(i, b)
        for b in range(NBUF):
            drain_write(b)

    return k


def kernel(x, table):
    B, H = x.shape
    V, D = table.shape
    N = B * H
    info = plsc.get_sparse_core_info()
    nw = info.num_cores * info.num_subcores
    n_per_w = N // nw
    k = _build(N, V, D, n_per_w, info.num_cores)
    out = k(x.reshape(N), table)
    return out.reshape(B, H, D)


# speedup vs baseline: 1.6283x; 1.6283x over previous
import functools

import jax
import jax.numpy as jnp
from jax import lax
from jax.experimental import pallas as pl
from jax.experimental.pallas import tpu as pltpu
from jax.experimental.pallas import tpu_sc as plsc

CHUNK = 400
NBUF = 8


def _build(B, H, V, D, n_per_w, num_cores):
    mesh = plsc.VectorSubcoreMesh(core_axis_name="c", subcore_axis_name="s")
    n_chunks = n_per_w // CHUNK
    rows_per_chunk = CHUNK // H
    assert CHUNK % H == 0 and n_chunks % NBUF == 0 and n_chunks >= 2 * NBUF

    @functools.partial(
        pl.kernel,
        mesh=mesh,
        out_type=jax.ShapeDtypeStruct((B, H, D), jnp.float32),
        scratch_types=[
            pltpu.VMEM((n_per_w,), jnp.int32),
            pltpu.VMEM((NBUF, CHUNK, D), jnp.float32),
            pltpu.SemaphoreType.DMA,
            pltpu.SemaphoreType.DMA,
        ],
        compiler_params=pltpu.CompilerParams(use_tc_tiling_on_sc=False),
    )
    def k(x_hbm, table_hbm, out_hbm, idx_v, rows_v, gsem, osem):
        wid = lax.axis_index("s") * num_cores + lax.axis_index("c")
        base = wid * n_per_w
        row_base = wid * (n_per_w // H)

        pltpu.sync_copy(x_hbm.at[pl.ds(base, n_per_w)], idx_v)

        def gather(i, b):
            return pltpu.async_copy(
                table_hbm.at[idx_v.at[pl.ds(i * CHUNK, CHUNK)]],
                rows_v.at[b],
                gsem,
            )

        def writeback(i, b):
            row0 = row_base + i * rows_per_chunk
            for r in range(rows_per_chunk):
                pltpu.async_copy(
                    rows_v.at[b, pl.ds(r * H, H), :],
                    out_hbm.at[row0 + r],
                    osem,
                )

        def drain_gather(b):
            pltpu.make_async_copy(
                table_hbm.at[idx_v.at[pl.ds(0, CHUNK)]], rows_v.at[b], gsem
            ).wait()

        def drain_write(b):
            pltpu.make_async_copy(
                table_hbm.at[pl.ds(0, CHUNK)], rows_v.at[b], osem
            ).wait()

        for b in range(NBUF):
            gather(b, b)

        def body(g, carry):
            for b in range(NBUF):
                i = g + b
                drain_gather(b)
                writeback(i, b)
                drain_write(b)
                gather(i + NBUF, b)
            return carry

        lax.fori_loop(0, (n_chunks - NBUF) // NBUF, lambda t, c: body(t * NBUF, c), 0)

        for b in range(NBUF):
            i = n_chunks - NBUF + b
            drain_gather(b)
            writeback(i, b)
        for b in range(NBUF):
            drain_write(b)

    return k


def kernel(x, table):
    B, H = x.shape
    V, D = table.shape
    N = B * H
    info = plsc.get_sparse_core_info()
    nw = info.num_cores * info.num_subcores
    n_per_w = N // nw
    k = _build(B, H, V, D, n_per_w, info.num_cores)
    return k(x.reshape(N), table)
